# baseline (speedup 1.0000x reference)
import functools

import jax
import jax.numpy as jnp
from jax.experimental import pallas as pl
from jax.experimental.pallas import tpu as pltpu

_B = 32
_M = 30
_SHAPES = [(64, 64, 64), (128, 32, 32), (256, 16, 16)]
_DS = [c * h * w for (c, h, w) in _SHAPES]
_NCHUNK = 8


def _dist_kernel(f1, m1, f2, m2, f3, m3, out_idx, acc):
    g = pl.program_id(0)

    @pl.when(g == 0)
    def _init():
        acc[:] = jnp.zeros_like(acc)

    for fr, mr, d in ((f1, m1, _DS[0]), (f2, m2, _DS[1]), (f3, m3, _DS[2])):
        a = fr[:]
        b = mr[:]
        cross = jax.lax.dot_general(
            a, b, (((1,), (1,)), ((), ())), preferred_element_type=jnp.float32
        )
        a2 = jnp.sum(a * a, axis=1)
        b2 = jnp.sum(b * b, axis=1)
        acc[:] += (a2[:, None] + b2[None, :] - 2.0 * cross) * (1.0 / d)

    @pl.when(g == _NCHUNK - 1)
    def _fin():
        out_idx[0, :] = jnp.argmin(acc[:], axis=1).astype(jnp.int32)


def _compute_idx(f1, m1, f2, m2, f3, m3):
    in_specs = []
    for d in _DS:
        ck = d // _NCHUNK
        in_specs.append(pl.BlockSpec((_B, ck), lambda i: (0, i)))
        in_specs.append(pl.BlockSpec((_M, ck), lambda i: (0, i)))
    idx = pl.pallas_call(
        _dist_kernel,
        grid=(_NCHUNK,),
        in_specs=in_specs,
        out_specs=pl.BlockSpec((1, _B), lambda i: (0, 0)),
        out_shape=jax.ShapeDtypeStruct((1, _B), jnp.int32),
        scratch_shapes=[pltpu.VMEM((_B, _M), jnp.float32)],
        compiler_params=pltpu.CompilerParams(
            dimension_semantics=("arbitrary",)
        ),
    )(f1, m1, f2, m2, f3, m3)
    return idx[0]


def _gather_kernel(idx_ref, f1, m1, f2, m2, f3, m3, o1, o2, o3):
    del idx_ref
    for f, m, o, (c, _, _) in (
        (f1, m1, o1, _SHAPES[0]),
        (f2, m2, o2, _SHAPES[1]),
        (f3, m3, o3, _SHAPES[2]),
    ):
        fv = f[0]
        mv = m[0]
        o[0, :c] = fv
        d = mv - fv
        o[0, c:] = d * d


def _compute_outputs(idx, f1, m1, f2, m2, f3, m3):
    in_specs = []
    out_specs = []
    out_shape = []
    for c, h, w in _SHAPES:
        in_specs.append(
            pl.BlockSpec((1, c, h * w), lambda b, idx_ref: (b, 0, 0))
        )
        in_specs.append(
            pl.BlockSpec((1, c, h * w), lambda b, idx_ref: (idx_ref[b], 0, 0))
        )
        out_specs.append(
            pl.BlockSpec((1, 2 * c, h * w), lambda b, idx_ref: (b, 0, 0))
        )
        out_shape.append(
            jax.ShapeDtypeStruct((_B, 2 * c, h * w), jnp.float32)
        )
    grid_spec = pltpu.PrefetchScalarGridSpec(
        num_scalar_prefetch=1,
        grid=(_B,),
        in_specs=in_specs,
        out_specs=out_specs,
    )
    return pl.pallas_call(
        _gather_kernel,
        grid_spec=grid_spec,
        out_shape=out_shape,
        compiler_params=pltpu.CompilerParams(
            dimension_semantics=("arbitrary",)
        ),
    )(idx, f1, m1, f2, m2, f3, m3)


@jax.jit
def kernel(feat1, feat2, feat3, mem1, mem2, mem3):
    feats = (feat1, feat2, feat3)
    mems = (mem1, mem2, mem3)
    ff = [f.reshape(_B, -1) for f in feats]
    mf = [m.reshape(_M, -1) for m in mems]
    idx = _compute_idx(ff[0], mf[0], ff[1], mf[1], ff[2], mf[2])

    f3d = [f.reshape(_B, c, h * w) for f, (c, h, w) in zip(feats, _SHAPES)]
    m3d = [m.reshape(_M, c, h * w) for m, (c, h, w) in zip(mems, _SHAPES)]
    outs = _compute_outputs(
        idx, f3d[0], m3d[0], f3d[1], m3d[1], f3d[2], m3d[2]
    )
    return tuple(
        o.reshape(_B, 2 * c, h, w) for o, (c, h, w) in zip(outs, _SHAPES)
    )
